# baseline (speedup 1.0000x reference)
import functools

import jax
import jax.numpy as jnp
from jax import lax
from jax.experimental import pallas as pl
from jax.experimental.pallas import tpu as pltpu
from jax.experimental.pallas import tpu_sc as plsc

N = 10000
E = 320000
D_ATOM = 128
D_EDGE = 16
HID = 64
HEADS = 8

NC = 2
NS = 16
TILES = NC * NS
EPT = E // TILES
EN = E + N
ES_PAD = 331776
NP_PAD = 10240
SEGS_PAD = NP_PAD + 16 * 9
NPT = 312
SLP = 336
EA_BLK = 2000

_mesh = plsc.VectorSubcoreMesh(core_axis_name="c", subcore_axis_name="s")
_sc_params = pltpu.CompilerParams(needs_layout_passes=False,
                                  use_tc_tiling_on_sc=False)

_DNUMS = lax.GatherDimensionNumbers(
    offset_dims=(), collapsed_slice_dims=(0,), start_index_map=(0,))


def _permute(x, idx):
    return lax.gather(x, idx[:, None], dimension_numbers=_DNUMS,
                      slice_sizes=(1,),
                      mode=lax.GatherScatterMode.PROMISE_IN_BOUNDS)


def _occ16(d16):
    lanes = lax.iota(jnp.int32, 16)
    occ = jnp.zeros((16,), jnp.int32)
    sh = d16
    for s in range(1, 16):
        sh = _permute(sh, jnp.maximum(lanes - 1, 0))
        occ = occ + jnp.where((sh == d16) & (lanes >= s), 1, 0)
    return occ


def _wid():
    return lax.axis_index("s") * NC + lax.axis_index("c")


@functools.partial(
    pl.kernel,
    out_type=(
        jax.ShapeDtypeStruct((TILES, NP_PAD), jnp.int32),
        jax.ShapeDtypeStruct((E,), jnp.int32),
        jax.ShapeDtypeStruct((NC, NS, N // NS, D_EDGE), jnp.float32),
    ),
    mesh=_mesh,
    scratch_types=(
        pltpu.VMEM((EPT,), jnp.int32),
        pltpu.VMEM((NP_PAD,), jnp.int32),
        pltpu.VMEM((EPT,), jnp.int32),
        pltpu.VMEM((EA_BLK, D_EDGE), jnp.float32),
        pltpu.VMEM((EA_BLK,), jnp.int32),
        pltpu.VMEM((N // NS, D_EDGE), jnp.float32),
        pltpu.VMEM_SHARED((N, D_EDGE), jnp.float32),
    ),
    compiler_params=_sc_params,
)
def _a1(dst_hbm, ea_hbm, hists_hbm, lrank_hbm, sege_hbm,
        dst_v, hist_v, lrank_v, ea_v, dstb_v, zb_v, sege_sh):
    wid = _wid()
    cid = lax.axis_index("c")
    sid = lax.axis_index("s")
    base = wid * EPT
    rows = N // NS

    def zrow(i, _):
        zb_v[i] = jnp.zeros((D_EDGE,), jnp.float32)
        return 0
    lax.fori_loop(0, rows, zrow, 0)
    pltpu.sync_copy(zb_v, sege_sh.at[pl.ds(sid * rows, rows)])

    def zhist(i, _):
        hist_v[pl.ds(i * 16, 16)] = jnp.zeros((16,), jnp.int32)
        return 0
    lax.fori_loop(0, NP_PAD // 16, zhist, 0)

    pltpu.sync_copy(dst_hbm.at[pl.ds(base, EPT)], dst_v)

    def body(i, _):
        sl = pl.ds(i * 16, 16)
        d16 = dst_v[sl]
        occ = _occ16(d16)
        c16 = plsc.load_gather(hist_v, [d16])
        lrank_v[sl] = c16 + occ
        plsc.addupdate_scatter(hist_v, [d16], jnp.ones((16,), jnp.int32))
        return 0
    lax.fori_loop(0, EPT // 16, body, 0)

    pltpu.sync_copy(hist_v, hists_hbm.at[wid])
    pltpu.sync_copy(lrank_v, lrank_hbm.at[pl.ds(base, EPT)])

    plsc.subcore_barrier()
    for b in range(EPT // EA_BLK):
        off = base + b * EA_BLK
        pltpu.sync_copy(ea_hbm.at[pl.ds(off, EA_BLK)], ea_v)
        pltpu.sync_copy(dst_hbm.at[pl.ds(off, EA_BLK)], dstb_v)
        pltpu.sync_copy(ea_v, sege_sh.at[dstb_v], add=True)
    plsc.subcore_barrier()

    pltpu.sync_copy(sege_sh.at[pl.ds(sid * rows, rows)], zb_v)
    pltpu.sync_copy(zb_v, sege_hbm.at[cid, sid])


_CH = 1024


@functools.partial(
    pl.kernel,
    out_type=(
        jax.ShapeDtypeStruct((TILES, NP_PAD), jnp.int32),
        jax.ShapeDtypeStruct((SEGS_PAD,), jnp.int32),
    ),
    mesh=_mesh,
    scratch_types=(
        pltpu.VMEM((TILES, _CH), jnp.int32),
        pltpu.VMEM((TILES, _CH), jnp.int32),
        pltpu.VMEM((_CH,), jnp.int32),
        pltpu.VMEM((16,), jnp.int32),
    ),
    compiler_params=_sc_params,
)
def _a2(hists_hbm, bases_hbm, segs_hbm, hcol_v, bcol_v, seg_v, pad_v):
    wid = _wid()

    @pl.when(wid == 0)
    def _():
        def chunk(ci, carry0):
            c0 = ci * _CH
            pltpu.sync_copy(hists_hbm.at[:, pl.ds(c0, _CH)], hcol_v)

            def step(j, carry_in):
                sl = pl.ds(j * 16, 16)
                tot = jnp.ones((16,), jnp.int32)
                for t in range(TILES):
                    tot = tot + hcol_v[t, sl]
                incl = plsc.cumsum(tot)
                seg = incl - tot + carry_in
                seg_v[sl] = seg
                b = seg + 1
                for t in range(TILES):
                    bcol_v[t, sl] = b
                    b = b + hcol_v[t, sl]
                return carry_in + jnp.sum(tot)

            carry1 = lax.fori_loop(0, _CH // 16, step, carry0)
            pltpu.sync_copy(bcol_v, bases_hbm.at[:, pl.ds(c0, _CH)])
            pltpu.sync_copy(seg_v, segs_hbm.at[pl.ds(c0, _CH)])
            return carry1

        lax.fori_loop(0, NP_PAD // _CH, chunk, jnp.int32(0))

        def pad(i, _):
            pad_v[...] = jnp.full((16,), EN, jnp.int32)
            pltpu.sync_copy(pad_v, segs_hbm.at[pl.ds(N + i * 16, 16)])
            return 0
        lax.fori_loop(0, (SEGS_PAD - N) // 16, pad, 0)


@functools.partial(
    pl.kernel,
    out_type=jax.ShapeDtypeStruct((NC, ES_PAD), jnp.int32),
    mesh=_mesh,
    scratch_types=(
        pltpu.VMEM((EPT,), jnp.int32),
        pltpu.VMEM((EPT,), jnp.int32),
        pltpu.VMEM((NP_PAD,), jnp.int32),
        pltpu.VMEM((EPT,), jnp.int32),
        pltpu.VMEM((EPT,), jnp.int32),
        pltpu.VMEM((SLP,), jnp.int32),
        pltpu.VMEM((SLP,), jnp.int32),
        pltpu.VMEM((ES_PAD // NS,), jnp.int32),
        pltpu.VMEM_SHARED((ES_PAD,), jnp.int32),
    ),
    compiler_params=_sc_params,
)
def _a3(dst_hbm, src_hbm, lrank_hbm, bases_hbm, segs_hbm, out_hbm,
        dst_v, lrank_v, bases_v, src_v, pos_v, spos_v, sval_v, zb_v,
        sorted_sh):
    wid = _wid()
    cid = lax.axis_index("c")
    sid = lax.axis_index("s")
    base = wid * EPT
    zwords = ES_PAD // NS

    def zrow(i, _):
        zb_v[pl.ds(i * 16, 16)] = jnp.zeros((16,), jnp.int32)
        return 0
    lax.fori_loop(0, zwords // 16, zrow, 0)
    pltpu.sync_copy(zb_v, sorted_sh.at[pl.ds(sid * zwords, zwords)])

    pltpu.sync_copy(dst_hbm.at[pl.ds(base, EPT)], dst_v)
    pltpu.sync_copy(lrank_hbm.at[pl.ds(base, EPT)], lrank_v)
    pltpu.sync_copy(bases_hbm.at[wid], bases_v)
    pltpu.sync_copy(src_hbm.at[pl.ds(base, EPT)], src_v)

    def mkpos(i, _):
        sl = pl.ds(i * 16, 16)
        d16 = dst_v[sl]
        b16 = plsc.load_gather(bases_v, [d16])
        pos_v[sl] = b16 + lrank_v[sl]
        return 0
    lax.fori_loop(0, EPT // 16, mkpos, 0)

    n0 = wid * NPT
    hi = jnp.where(wid == TILES - 1, N, n0 + NPT)
    pltpu.sync_copy(segs_hbm.at[pl.ds(n0, SLP)], spos_v)

    def mkself(i, _):
        sl = pl.ds(i * 16, 16)
        node = lax.iota(jnp.int32, 16) + (n0 + i * 16)
        ok = node < hi
        sval_v[sl] = jnp.where(ok, node, 0)
        spos_v[sl] = jnp.where(ok, spos_v[sl], EN)
        return 0
    lax.fori_loop(0, SLP // 16, mkself, 0)

    plsc.subcore_barrier()
    pltpu.sync_copy(src_v, sorted_sh.at[pos_v], add=True)
    pltpu.sync_copy(sval_v, sorted_sh.at[spos_v], add=True)
    plsc.subcore_barrier()

    pltpu.sync_copy(sorted_sh.at[pl.ds(sid * zwords, zwords)], zb_v)
    pltpu.sync_copy(zb_v, out_hbm.at[cid, pl.ds(sid * zwords, zwords)])


def _bn(x, g, b):
    mu = jnp.mean(x, axis=0)
    var = jnp.var(x, axis=0)
    return (x - mu) * jax.lax.rsqrt(var + 1e-5) * g + b


def _post_kernel(gat_ref, gat_b_ref, g_bn_ref, be_bn_ref, W_p1_ref, b_p1_ref,
                 g_p_ref, be_p_ref, W_p2_ref, b_p2_ref, out_ref):
    gat = gat_ref[...] + gat_b_ref[...]
    h = jax.nn.relu(_bn(gat, g_bn_ref[...], be_bn_ref[...]))
    h2 = jax.nn.relu(_bn(h @ W_p1_ref[...] + b_p1_ref[...], g_p_ref[...], be_p_ref[...]))
    out_ref[...] = (h2 @ W_p2_ref[...] + b_p2_ref[...])


def kernel(x, edge_index, edge_attr, W_atom, b_atom, W_edge, b_edge, W_msg, b_msg, g_msg, be_msg, W_l, b_l, W_r, b_r, att, gat_b, g_bn, be_bn, W_p1, b_p1, g_p, be_p, W_p2, b_p2):
    src = edge_index[0]
    dst = edge_index[1]

    hists, lrank, sege2 = _a1(dst, edge_attr)
    bases, segs = _a2(hists)
    ss2 = _a3(dst, src, lrank, bases, segs)

    src_sorted = (ss2[0] + ss2[1])[:EN]
    sege = (sege2[0] + sege2[1]).reshape(N, D_EDGE)
    cnt = hists[:, :N].sum(0).astype(jnp.float32)
    seg_start = segs[:N]

    atom = x @ W_atom + b_atom
    agg = (sege @ W_edge + cnt[:, None] * b_edge) / jnp.maximum(cnt, 1.0)[:, None]
    msg = jax.nn.relu(_bn((atom + agg) @ W_msg + b_msg, g_msg, be_msg))
    comb = jnp.concatenate([msg, agg], axis=1)
    xl = (comb @ W_l + b_l).reshape(N, HEADS, HID)
    xr = (comb @ W_r + b_r).reshape(N, HEADS, HID)

    seg_len = jnp.diff(jnp.concatenate([seg_start, jnp.array([EN], jnp.int32)]))
    d_sorted = jnp.repeat(jnp.arange(N, dtype=jnp.int32), seg_len,
                          total_repeat_length=EN)
    x_j = xl[src_sorted]
    x_i = xr[d_sorted]
    e = jax.nn.leaky_relu(x_i + x_j, 0.2)
    alpha = jnp.sum(e * att, axis=-1)
    p = jnp.exp(alpha)
    denom = jax.ops.segment_sum(p, d_sorted, num_segments=N)
    w = p / (denom[d_sorted] + 1e-16)
    gat = jax.ops.segment_sum(x_j * w[:, :, None], d_sorted, num_segments=N)
    gat = gat.reshape(N, HEADS * HID)

    out2 = pl.pallas_call(
        _post_kernel,
        out_shape=jax.ShapeDtypeStruct((N, 1), jnp.float32),
    )(gat, gat_b, g_bn, be_bn, W_p1, b_p1, g_p, be_p, W_p2, b_p2)
    return out2[:, 0]


# speedup vs baseline: 19.0419x; 19.0419x over previous
import functools

import jax
import jax.numpy as jnp
from jax import lax
from jax.experimental import pallas as pl
from jax.experimental.pallas import tpu as pltpu
from jax.experimental.pallas import tpu_sc as plsc

N = 10000
E = 320000
D_ATOM = 128
D_EDGE = 16
HID = 64
HEADS = 8

NC = 2
NS = 16
TILES = NC * NS
EPT = E // TILES
EN = E + N
ES_PAD = 333056
NP_PAD = 10240
SEGS_PAD = NP_PAD + 16 * 9
NPT = 312
SLP = 336
EA_BLK = 2000
DL = HEADS * HID
CAP = 12288
NTA = 320
NTB = 304

_mesh = plsc.VectorSubcoreMesh(core_axis_name="c", subcore_axis_name="s")
_sc_params = pltpu.CompilerParams(needs_layout_passes=False,
                                  use_tc_tiling_on_sc=False)

_DNUMS = lax.GatherDimensionNumbers(
    offset_dims=(), collapsed_slice_dims=(0,), start_index_map=(0,))


def _permute(x, idx):
    return lax.gather(x, idx[:, None], dimension_numbers=_DNUMS,
                      slice_sizes=(1,),
                      mode=lax.GatherScatterMode.PROMISE_IN_BOUNDS)


def _occ16(d16):
    lanes = lax.iota(jnp.int32, 16)
    occ = jnp.zeros((16,), jnp.int32)
    sh = d16
    for s in range(1, 16):
        sh = _permute(sh, jnp.maximum(lanes - 1, 0))
        occ = occ + jnp.where((sh == d16) & (lanes >= s), 1, 0)
    return occ


def _wid():
    return lax.axis_index("s") * NC + lax.axis_index("c")


@functools.partial(
    pl.kernel,
    out_type=(
        jax.ShapeDtypeStruct((TILES, NP_PAD), jnp.int32),
        jax.ShapeDtypeStruct((E,), jnp.int32),
        jax.ShapeDtypeStruct((NC, NS, N // NS, D_EDGE), jnp.float32),
    ),
    mesh=_mesh,
    scratch_types=(
        pltpu.VMEM((EPT,), jnp.int32),
        pltpu.VMEM((NP_PAD,), jnp.int32),
        pltpu.VMEM((EPT,), jnp.int32),
        pltpu.VMEM((EA_BLK, D_EDGE), jnp.float32),
        pltpu.VMEM((EA_BLK,), jnp.int32),
        pltpu.VMEM((N // NS, D_EDGE), jnp.float32),
        pltpu.VMEM_SHARED((N, D_EDGE), jnp.float32),
    ),
    compiler_params=_sc_params,
)
def _a1(dst_hbm, ea_hbm, hists_hbm, lrank_hbm, sege_hbm,
        dst_v, hist_v, lrank_v, ea_v, dstb_v, zb_v, sege_sh):
    wid = _wid()
    cid = lax.axis_index("c")
    sid = lax.axis_index("s")
    base = wid * EPT
    rows = N // NS

    def zrow(i, _):
        zb_v[i] = jnp.zeros((D_EDGE,), jnp.float32)
        return 0
    lax.fori_loop(0, rows, zrow, 0)
    pltpu.sync_copy(zb_v, sege_sh.at[pl.ds(sid * rows, rows)])

    def zhist(i, _):
        hist_v[pl.ds(i * 16, 16)] = jnp.zeros((16,), jnp.int32)
        return 0
    lax.fori_loop(0, NP_PAD // 16, zhist, 0)

    pltpu.sync_copy(dst_hbm.at[pl.ds(base, EPT)], dst_v)

    def body(i, _):
        sl = pl.ds(i * 16, 16)
        d16 = dst_v[sl]
        occ = _occ16(d16)
        c16 = plsc.load_gather(hist_v, [d16])
        lrank_v[sl] = c16 + occ
        plsc.addupdate_scatter(hist_v, [d16], jnp.ones((16,), jnp.int32))
        return 0
    lax.fori_loop(0, EPT // 16, body, 0)

    pltpu.sync_copy(hist_v, hists_hbm.at[wid])
    pltpu.sync_copy(lrank_v, lrank_hbm.at[pl.ds(base, EPT)])

    plsc.subcore_barrier()
    for b in range(EPT // EA_BLK):
        off = base + b * EA_BLK
        pltpu.sync_copy(ea_hbm.at[pl.ds(off, EA_BLK)], ea_v)
        pltpu.sync_copy(dst_hbm.at[pl.ds(off, EA_BLK)], dstb_v)
        pltpu.sync_copy(ea_v, sege_sh.at[dstb_v], add=True)
    plsc.subcore_barrier()

    pltpu.sync_copy(sege_sh.at[pl.ds(sid * rows, rows)], zb_v)
    pltpu.sync_copy(zb_v, sege_hbm.at[cid, sid])


_CH = 1024


@functools.partial(
    pl.kernel,
    out_type=(
        jax.ShapeDtypeStruct((TILES, NP_PAD), jnp.int32),
        jax.ShapeDtypeStruct((SEGS_PAD,), jnp.int32),
    ),
    mesh=_mesh,
    scratch_types=(
        pltpu.VMEM((TILES, _CH), jnp.int32),
        pltpu.VMEM((TILES, _CH), jnp.int32),
        pltpu.VMEM((_CH,), jnp.int32),
        pltpu.VMEM((16,), jnp.int32),
    ),
    compiler_params=_sc_params,
)
def _a2(hists_hbm, bases_hbm, segs_hbm, hcol_v, bcol_v, seg_v, pad_v):
    wid = _wid()

    @pl.when(wid == 0)
    def _():
        def chunk(ci, carry0):
            c0 = ci * _CH
            pltpu.sync_copy(hists_hbm.at[:, pl.ds(c0, _CH)], hcol_v)

            def step(j, carry_in):
                sl = pl.ds(j * 16, 16)
                tot = jnp.ones((16,), jnp.int32)
                for t in range(TILES):
                    tot = tot + hcol_v[t, sl]
                incl = plsc.cumsum(tot)
                seg = incl - tot + carry_in
                seg_v[sl] = seg
                b = seg + 1
                for t in range(TILES):
                    bcol_v[t, sl] = b
                    b = b + hcol_v[t, sl]
                return carry_in + jnp.sum(tot)

            carry1 = lax.fori_loop(0, _CH // 16, step, carry0)
            pltpu.sync_copy(bcol_v, bases_hbm.at[:, pl.ds(c0, _CH)])
            pltpu.sync_copy(seg_v, segs_hbm.at[pl.ds(c0, _CH)])
            return carry1

        lax.fori_loop(0, NP_PAD // _CH, chunk, jnp.int32(0))

        def pad(i, _):
            pad_v[...] = jnp.full((16,), EN, jnp.int32)
            pltpu.sync_copy(pad_v, segs_hbm.at[pl.ds(N + i * 16, 16)])
            return 0
        lax.fori_loop(0, (SEGS_PAD - N) // 16, pad, 0)


@functools.partial(
    pl.kernel,
    out_type=jax.ShapeDtypeStruct((NC, ES_PAD), jnp.int32),
    mesh=_mesh,
    scratch_types=(
        pltpu.VMEM((EPT,), jnp.int32),
        pltpu.VMEM((EPT,), jnp.int32),
        pltpu.VMEM((NP_PAD,), jnp.int32),
        pltpu.VMEM((EPT,), jnp.int32),
        pltpu.VMEM((EPT,), jnp.int32),
        pltpu.VMEM((SLP,), jnp.int32),
        pltpu.VMEM((SLP,), jnp.int32),
        pltpu.VMEM((ES_PAD // NS,), jnp.int32),
        pltpu.VMEM_SHARED((ES_PAD,), jnp.int32),
    ),
    compiler_params=_sc_params,
)
def _a3(dst_hbm, src_hbm, lrank_hbm, bases_hbm, segs_hbm, out_hbm,
        dst_v, lrank_v, bases_v, src_v, pos_v, spos_v, sval_v, zb_v,
        sorted_sh):
    wid = _wid()
    cid = lax.axis_index("c")
    sid = lax.axis_index("s")
    base = wid * EPT
    zwords = ES_PAD // NS

    def zrow(i, _):
        zb_v[pl.ds(i * 16, 16)] = jnp.zeros((16,), jnp.int32)
        return 0
    lax.fori_loop(0, zwords // 16, zrow, 0)
    pltpu.sync_copy(zb_v, sorted_sh.at[pl.ds(sid * zwords, zwords)])

    pltpu.sync_copy(dst_hbm.at[pl.ds(base, EPT)], dst_v)
    pltpu.sync_copy(lrank_hbm.at[pl.ds(base, EPT)], lrank_v)
    pltpu.sync_copy(bases_hbm.at[wid], bases_v)
    pltpu.sync_copy(src_hbm.at[pl.ds(base, EPT)], src_v)

    def mkpos(i, _):
        sl = pl.ds(i * 16, 16)
        d16 = dst_v[sl]
        b16 = plsc.load_gather(bases_v, [d16])
        pos_v[sl] = b16 + lrank_v[sl]
        return 0
    lax.fori_loop(0, EPT // 16, mkpos, 0)

    n0 = wid * NPT
    hi = jnp.where(wid == TILES - 1, N, n0 + NPT)
    pltpu.sync_copy(segs_hbm.at[pl.ds(n0, SLP)], spos_v)

    def mkself(i, _):
        sl = pl.ds(i * 16, 16)
        node = lax.iota(jnp.int32, 16) + (n0 + i * 16)
        ok = node < hi
        sval_v[sl] = jnp.where(ok, node, 0)
        spos_v[sl] = jnp.where(ok, spos_v[sl], EN)
        return 0
    lax.fori_loop(0, SLP // 16, mkself, 0)

    plsc.subcore_barrier()
    pltpu.sync_copy(src_v, sorted_sh.at[pos_v], add=True)
    pltpu.sync_copy(sval_v, sorted_sh.at[spos_v], add=True)
    plsc.subcore_barrier()

    pltpu.sync_copy(sorted_sh.at[pl.ds(sid * zwords, zwords)], zb_v)
    pltpu.sync_copy(zb_v, out_hbm.at[cid, pl.ds(sid * zwords, zwords)])


@functools.partial(
    pl.kernel,
    out_type=jax.ShapeDtypeStruct((N, DL), jnp.float32),
    mesh=_mesh,
    scratch_types=(
        pltpu.VMEM((SEGS_PAD,), jnp.int32),
        pltpu.VMEM((CAP,), jnp.int32),
        pltpu.VMEM((DL,), jnp.float32),
        pltpu.VMEM((16, DL), jnp.float32),
        pltpu.VMEM((16, DL), jnp.float32),
        pltpu.VMEM((DL,), jnp.float32),
        pltpu.VMEM((HEADS, 16), jnp.float32),
        pltpu.SemaphoreType.DMA,
    ),
    compiler_params=_sc_params,
)
def _a4(xl_hbm, xr_hbm, srcs_hbm, segs_hbm, att_hbm, gat_hbm,
        seg_v, span_v, att_v, xrg_v, rows_v, acc_v, den_v, sem):
    wid = _wid()
    n0 = jnp.where(wid < 17, wid * NTA, 17 * NTA + (wid - 17) * NTB)
    nn = jnp.where(wid < 17, NTA, NTB)
    lanes = lax.iota(jnp.int32, 16)

    pltpu.sync_copy(segs_hbm, seg_v)
    pltpu.sync_copy(att_hbm, att_v)
    e0 = seg_v[pl.ds(n0, 16)][0]
    ebase = jnp.minimum((e0 // 8) * 8, ES_PAD - CAP)

    def cpspan(k, _):
        pltpu.sync_copy(srcs_hbm.at[pl.ds(ebase + k * 2048, 2048)],
                        span_v.at[pl.ds(k * 2048, 2048)])
        return 0
    lax.fori_loop(0, CAP // 2048, cpspan, 0)

    def group(gi, _):
        base = n0 + gi * 16
        sva = seg_v[pl.ds(base, 16)]
        send = seg_v[pl.ds(base + 8, 16)][8]
        pltpu.sync_copy(xr_hbm.at[pl.ds(base, 16)], xrg_v)

        def node(rr, _):
            cur = jnp.sum(jnp.where(lanes == rr, sva, 0))
            nxt = jnp.where(rr == 15, send,
                            jnp.sum(jnp.where(lanes == rr + 1, sva, 0)))

            def zc(c, _):
                acc_v[pl.ds(c * 16, 16)] = jnp.zeros((16,), jnp.float32)
                return 0
            lax.fori_loop(0, DL // 16, zc, 0)

            def zd(h, _):
                den_v[h, :] = jnp.zeros((16,), jnp.float32)
                return 0
            lax.fori_loop(0, HEADS, zd, 0)

            nch = (jnp.minimum(nxt - cur, CAP) + 15) // 16

            def chunk(ch, _):
                pos = cur + ch * 16 + lanes
                valid = pos < nxt
                rel = jnp.clip(jnp.where(valid, pos, nxt - 1) - ebase,
                               0, CAP - 1)
                idx16 = plsc.load_gather(span_v, [rel])
                pltpu.async_copy(xl_hbm.at[idx16], rows_v, sem).wait()

                def head(h, _):
                    logit = jnp.zeros((16,), jnp.float32)
                    for r in range(16):
                        def af(c, p):
                            sl = pl.ds(h * HID + c * 16, 16)
                            t = rows_v[r, sl] + xrg_v[rr, sl]
                            lr = 0.6 * t + 0.4 * jnp.abs(t)
                            return p + lr * att_v[sl]
                        p = lax.fori_loop(0, HID // 16, af,
                                          jnp.zeros((16,), jnp.float32))
                        logit = logit + jnp.where(lanes == r, jnp.sum(p), 0.0)
                    w = jnp.where(valid, jnp.exp(logit), 0.0)
                    den_v[h, :] = den_v[h, :] + w
                    for r in range(16):
                        wr = w[r]

                        def cf(c, _):
                            sl = pl.ds(h * HID + c * 16, 16)
                            acc_v[sl] = acc_v[sl] + wr * rows_v[r, sl]
                            return 0
                        lax.fori_loop(0, HID // 16, cf, 0)
                    return 0
                lax.fori_loop(0, HEADS, head, 0)
                return 0
            lax.fori_loop(0, nch, chunk, 0)

            def norm(h, _):
                dsum = jnp.sum(den_v[h, :]) + 1e-16
                rinv = 1.0 / jnp.full((16,), dsum, jnp.float32)

                def nf(c, _):
                    sl = pl.ds(h * HID + c * 16, 16)
                    acc_v[sl] = acc_v[sl] * rinv
                    return 0
                lax.fori_loop(0, HID // 16, nf, 0)
                return 0
            lax.fori_loop(0, HEADS, norm, 0)

            pltpu.sync_copy(acc_v, gat_hbm.at[base + rr])
            return 0
        lax.fori_loop(0, 16, node, 0)
        return 0
    lax.fori_loop(0, nn // 16, group, 0)


def _bn(x, g, b):
    mu = jnp.mean(x, axis=0)
    var = jnp.var(x, axis=0)
    return (x - mu) * jax.lax.rsqrt(var + 1e-5) * g + b


def _post_kernel(gat_ref, gat_b_ref, g_bn_ref, be_bn_ref, W_p1_ref, b_p1_ref,
                 g_p_ref, be_p_ref, W_p2_ref, b_p2_ref, out_ref):
    gat = gat_ref[...] + gat_b_ref[...]
    h = jax.nn.relu(_bn(gat, g_bn_ref[...], be_bn_ref[...]))
    h2 = jax.nn.relu(_bn(h @ W_p1_ref[...] + b_p1_ref[...], g_p_ref[...], be_p_ref[...]))
    out_ref[...] = (h2 @ W_p2_ref[...] + b_p2_ref[...])


def kernel(x, edge_index, edge_attr, W_atom, b_atom, W_edge, b_edge, W_msg, b_msg, g_msg, be_msg, W_l, b_l, W_r, b_r, att, gat_b, g_bn, be_bn, W_p1, b_p1, g_p, be_p, W_p2, b_p2):
    src = edge_index[0]
    dst = edge_index[1]

    hists, lrank, sege2 = _a1(dst, edge_attr)
    bases, segs = _a2(hists)
    ss2 = _a3(dst, src, lrank, bases, segs)

    srcs = ss2[0] + ss2[1]
    sege = (sege2[0] + sege2[1]).reshape(N, D_EDGE)
    cnt = hists[:, :N].sum(0).astype(jnp.float32)

    atom = x @ W_atom + b_atom
    agg = (sege @ W_edge + cnt[:, None] * b_edge) / jnp.maximum(cnt, 1.0)[:, None]
    msg = jax.nn.relu(_bn((atom + agg) @ W_msg + b_msg, g_msg, be_msg))
    comb = jnp.concatenate([msg, agg], axis=1)
    xl = comb @ W_l + b_l
    xr = comb @ W_r + b_r

    gat = _a4(xl, xr, srcs, segs, att.reshape(-1))

    out2 = pl.pallas_call(
        _post_kernel,
        out_shape=jax.ShapeDtypeStruct((N, 1), jnp.float32),
    )(gat, gat_b, g_bn, be_bn, W_p1, b_p1, g_p, be_p, W_p2, b_p2)
    return out2[:, 0]


# speedup vs baseline: 20.6501x; 1.0845x over previous
import functools

import jax
import jax.numpy as jnp
from jax import lax
from jax.experimental import pallas as pl
from jax.experimental.pallas import tpu as pltpu
from jax.experimental.pallas import tpu_sc as plsc

N = 10000
E = 320000
D_ATOM = 128
D_EDGE = 16
HID = 64
HEADS = 8

NC = 2
NS = 16
TILES = NC * NS
EPT = E // TILES
EN = E + N
ES_PAD = 333056
NP_PAD = 10240
SEGS_PAD = NP_PAD + 16 * 9
NPT = 312
SLP = 336
EA_BLK = 2000
DL = HEADS * HID
CAP = 12288
NTA = 320
NTB = 304

_mesh = plsc.VectorSubcoreMesh(core_axis_name="c", subcore_axis_name="s")
_sc_params = pltpu.CompilerParams(needs_layout_passes=False,
                                  use_tc_tiling_on_sc=False)

_DNUMS = lax.GatherDimensionNumbers(
    offset_dims=(), collapsed_slice_dims=(0,), start_index_map=(0,))


def _permute(x, idx):
    return lax.gather(x, idx[:, None], dimension_numbers=_DNUMS,
                      slice_sizes=(1,),
                      mode=lax.GatherScatterMode.PROMISE_IN_BOUNDS)


def _occ16(d16):
    lanes = lax.iota(jnp.int32, 16)
    occ = jnp.zeros((16,), jnp.int32)
    sh = d16
    for s in range(1, 16):
        sh = _permute(sh, jnp.maximum(lanes - 1, 0))
        occ = occ + jnp.where((sh == d16) & (lanes >= s), 1, 0)
    return occ


def _wid():
    return lax.axis_index("s") * NC + lax.axis_index("c")


@functools.partial(
    pl.kernel,
    out_type=(
        jax.ShapeDtypeStruct((TILES, NP_PAD), jnp.int32),
        jax.ShapeDtypeStruct((E,), jnp.int32),
        jax.ShapeDtypeStruct((NC, NS, N // NS, D_EDGE), jnp.float32),
    ),
    mesh=_mesh,
    scratch_types=(
        pltpu.VMEM((EPT,), jnp.int32),
        pltpu.VMEM((NP_PAD,), jnp.int32),
        pltpu.VMEM((EPT,), jnp.int32),
        pltpu.VMEM((EA_BLK, D_EDGE), jnp.float32),
        pltpu.VMEM((EA_BLK,), jnp.int32),
        pltpu.VMEM((N // NS, D_EDGE), jnp.float32),
        pltpu.VMEM_SHARED((N, D_EDGE), jnp.float32),
    ),
    compiler_params=_sc_params,
)
def _a1(dst_hbm, ea_hbm, hists_hbm, lrank_hbm, sege_hbm,
        dst_v, hist_v, lrank_v, ea_v, dstb_v, zb_v, sege_sh):
    wid = _wid()
    cid = lax.axis_index("c")
    sid = lax.axis_index("s")
    base = wid * EPT
    rows = N // NS

    def zrow(i, _):
        zb_v[i] = jnp.zeros((D_EDGE,), jnp.float32)
        return 0
    lax.fori_loop(0, rows, zrow, 0)
    pltpu.sync_copy(zb_v, sege_sh.at[pl.ds(sid * rows, rows)])

    def zhist(i, _):
        hist_v[pl.ds(i * 16, 16)] = jnp.zeros((16,), jnp.int32)
        return 0
    lax.fori_loop(0, NP_PAD // 16, zhist, 0)

    pltpu.sync_copy(dst_hbm.at[pl.ds(base, EPT)], dst_v)

    def body(i, _):
        sl = pl.ds(i * 16, 16)
        d16 = dst_v[sl]
        occ = _occ16(d16)
        c16 = plsc.load_gather(hist_v, [d16])
        lrank_v[sl] = c16 + occ
        plsc.addupdate_scatter(hist_v, [d16], jnp.ones((16,), jnp.int32))
        return 0
    lax.fori_loop(0, EPT // 16, body, 0)

    pltpu.sync_copy(hist_v, hists_hbm.at[wid])
    pltpu.sync_copy(lrank_v, lrank_hbm.at[pl.ds(base, EPT)])

    plsc.subcore_barrier()
    for b in range(EPT // EA_BLK):
        off = base + b * EA_BLK
        pltpu.sync_copy(ea_hbm.at[pl.ds(off, EA_BLK)], ea_v)
        pltpu.sync_copy(dst_hbm.at[pl.ds(off, EA_BLK)], dstb_v)
        pltpu.sync_copy(ea_v, sege_sh.at[dstb_v], add=True)
    plsc.subcore_barrier()

    pltpu.sync_copy(sege_sh.at[pl.ds(sid * rows, rows)], zb_v)
    pltpu.sync_copy(zb_v, sege_hbm.at[cid, sid])


_CH = 1024


@functools.partial(
    pl.kernel,
    out_type=(
        jax.ShapeDtypeStruct((TILES, NP_PAD), jnp.int32),
        jax.ShapeDtypeStruct((SEGS_PAD,), jnp.int32),
    ),
    mesh=_mesh,
    scratch_types=(
        pltpu.VMEM((TILES, _CH), jnp.int32),
        pltpu.VMEM((TILES, _CH), jnp.int32),
        pltpu.VMEM((_CH,), jnp.int32),
        pltpu.VMEM((16,), jnp.int32),
    ),
    compiler_params=_sc_params,
)
def _a2(hists_hbm, bases_hbm, segs_hbm, hcol_v, bcol_v, seg_v, pad_v):
    wid = _wid()

    @pl.when(wid == 0)
    def _():
        def chunk(ci, carry0):
            c0 = ci * _CH
            pltpu.sync_copy(hists_hbm.at[:, pl.ds(c0, _CH)], hcol_v)

            def step(j, carry_in):
                sl = pl.ds(j * 16, 16)
                tot = jnp.ones((16,), jnp.int32)
                for t in range(TILES):
                    tot = tot + hcol_v[t, sl]
                incl = plsc.cumsum(tot)
                seg = incl - tot + carry_in
                seg_v[sl] = seg
                b = seg + 1
                for t in range(TILES):
                    bcol_v[t, sl] = b
                    b = b + hcol_v[t, sl]
                return carry_in + jnp.sum(tot)

            carry1 = lax.fori_loop(0, _CH // 16, step, carry0)
            pltpu.sync_copy(bcol_v, bases_hbm.at[:, pl.ds(c0, _CH)])
            pltpu.sync_copy(seg_v, segs_hbm.at[pl.ds(c0, _CH)])
            return carry1

        lax.fori_loop(0, NP_PAD // _CH, chunk, jnp.int32(0))

        def pad(i, _):
            pad_v[...] = jnp.full((16,), EN, jnp.int32)
            pltpu.sync_copy(pad_v, segs_hbm.at[pl.ds(N + i * 16, 16)])
            return 0
        lax.fori_loop(0, (SEGS_PAD - N) // 16, pad, 0)


@functools.partial(
    pl.kernel,
    out_type=jax.ShapeDtypeStruct((NC, ES_PAD), jnp.int32),
    mesh=_mesh,
    scratch_types=(
        pltpu.VMEM((EPT,), jnp.int32),
        pltpu.VMEM((EPT,), jnp.int32),
        pltpu.VMEM((NP_PAD,), jnp.int32),
        pltpu.VMEM((EPT,), jnp.int32),
        pltpu.VMEM((EPT,), jnp.int32),
        pltpu.VMEM((SLP,), jnp.int32),
        pltpu.VMEM((SLP,), jnp.int32),
        pltpu.VMEM((ES_PAD // NS,), jnp.int32),
        pltpu.VMEM_SHARED((ES_PAD,), jnp.int32),
    ),
    compiler_params=_sc_params,
)
def _a3(dst_hbm, src_hbm, lrank_hbm, bases_hbm, segs_hbm, out_hbm,
        dst_v, lrank_v, bases_v, src_v, pos_v, spos_v, sval_v, zb_v,
        sorted_sh):
    wid = _wid()
    cid = lax.axis_index("c")
    sid = lax.axis_index("s")
    base = wid * EPT
    zwords = ES_PAD // NS

    def zrow(i, _):
        zb_v[pl.ds(i * 16, 16)] = jnp.zeros((16,), jnp.int32)
        return 0
    lax.fori_loop(0, zwords // 16, zrow, 0)
    pltpu.sync_copy(zb_v, sorted_sh.at[pl.ds(sid * zwords, zwords)])

    pltpu.sync_copy(dst_hbm.at[pl.ds(base, EPT)], dst_v)
    pltpu.sync_copy(lrank_hbm.at[pl.ds(base, EPT)], lrank_v)
    pltpu.sync_copy(bases_hbm.at[wid], bases_v)
    pltpu.sync_copy(src_hbm.at[pl.ds(base, EPT)], src_v)

    def mkpos(i, _):
        sl = pl.ds(i * 16, 16)
        d16 = dst_v[sl]
        b16 = plsc.load_gather(bases_v, [d16])
        pos_v[sl] = b16 + lrank_v[sl]
        return 0
    lax.fori_loop(0, EPT // 16, mkpos, 0)

    n0 = wid * NPT
    hi = jnp.where(wid == TILES - 1, N, n0 + NPT)
    pltpu.sync_copy(segs_hbm.at[pl.ds(n0, SLP)], spos_v)

    def mkself(i, _):
        sl = pl.ds(i * 16, 16)
        node = lax.iota(jnp.int32, 16) + (n0 + i * 16)
        ok = node < hi
        sval_v[sl] = jnp.where(ok, node, 0)
        spos_v[sl] = jnp.where(ok, spos_v[sl], EN)
        return 0
    lax.fori_loop(0, SLP // 16, mkself, 0)

    plsc.subcore_barrier()
    pltpu.sync_copy(src_v, sorted_sh.at[pos_v], add=True)
    pltpu.sync_copy(sval_v, sorted_sh.at[spos_v], add=True)
    plsc.subcore_barrier()

    pltpu.sync_copy(sorted_sh.at[pl.ds(sid * zwords, zwords)], zb_v)
    pltpu.sync_copy(zb_v, out_hbm.at[cid, pl.ds(sid * zwords, zwords)])


@functools.partial(
    pl.kernel,
    out_type=jax.ShapeDtypeStruct((N, DL), jnp.float32),
    mesh=_mesh,
    scratch_types=(
        pltpu.VMEM((SEGS_PAD,), jnp.int32),
        pltpu.VMEM((CAP,), jnp.int32),
        pltpu.VMEM((DL,), jnp.float32),
        pltpu.VMEM((16, DL), jnp.float32),
        pltpu.VMEM((16, DL), jnp.float32),
        pltpu.VMEM((DL,), jnp.float32),
        pltpu.VMEM((HEADS, 16), jnp.float32),
        pltpu.SemaphoreType.DMA,
    ),
    compiler_params=_sc_params,
)
def _a4(xl_hbm, xr_hbm, srcs_hbm, segs_hbm, att_hbm, gat_hbm,
        seg_v, span_v, att_v, xrg_v, rows_v, acc_v, den_v, sem):
    wid = _wid()
    n0 = jnp.where(wid < 17, wid * NTA, 17 * NTA + (wid - 17) * NTB)
    nn = jnp.where(wid < 17, NTA, NTB)
    lanes = lax.iota(jnp.int32, 16)

    pltpu.sync_copy(segs_hbm, seg_v)
    pltpu.sync_copy(att_hbm, att_v)
    e0 = seg_v[pl.ds(n0, 16)][0]
    ebase = jnp.minimum((e0 // 8) * 8, ES_PAD - CAP)

    def cpspan(k, _):
        pltpu.sync_copy(srcs_hbm.at[pl.ds(ebase + k * 2048, 2048)],
                        span_v.at[pl.ds(k * 2048, 2048)])
        return 0
    lax.fori_loop(0, CAP // 2048, cpspan, 0)

    def group(gi, _):
        base = n0 + gi * 16
        sva = seg_v[pl.ds(base, 16)]
        send = seg_v[pl.ds(base + 8, 16)][8]
        pltpu.sync_copy(xr_hbm.at[pl.ds(base, 16)], xrg_v)

        def node(rr, _):
            cur = jnp.sum(jnp.where(lanes == rr, sva, 0))
            nxt = jnp.where(rr == 15, send,
                            jnp.sum(jnp.where(lanes == rr + 1, sva, 0)))

            def zc(c, _):
                acc_v[pl.ds(c * 16, 16)] = jnp.zeros((16,), jnp.float32)
                return 0
            lax.fori_loop(0, DL // 16, zc, 0)

            def zd(h, _):
                den_v[h, :] = jnp.zeros((16,), jnp.float32)
                return 0
            lax.fori_loop(0, HEADS, zd, 0)

            nch = (jnp.minimum(nxt - cur, CAP) + 15) // 16

            def chunk(ch, _):
                pos = cur + ch * 16 + lanes
                valid = pos < nxt
                rel = jnp.clip(jnp.where(valid, pos, nxt - 1) - ebase,
                               0, CAP - 1)
                idx16 = plsc.load_gather(span_v, [rel])
                pltpu.async_copy(xl_hbm.at[idx16], rows_v, sem).wait()

                def head(h, _):
                    sls = [pl.ds(h * HID + c * 16, 16) for c in range(HID // 16)]
                    xrh = [xrg_v[rr, sl] for sl in sls]
                    ath = [att_v[sl] for sl in sls]
                    logit = jnp.zeros((16,), jnp.float32)
                    for r in range(16):
                        p = jnp.zeros((16,), jnp.float32)
                        for c, sl in enumerate(sls):
                            t = rows_v[r, sl] + xrh[c]
                            p = p + (0.6 * t + 0.4 * jnp.abs(t)) * ath[c]
                        logit = logit + jnp.where(lanes == r, jnp.sum(p), 0.0)
                    w = jnp.where(valid, jnp.exp(logit), 0.0)
                    den_v[h, :] = den_v[h, :] + w
                    for c, sl in enumerate(sls):
                        a = acc_v[sl]
                        for r in range(16):
                            a = a + w[r] * rows_v[r, sl]
                        acc_v[sl] = a
                    return 0
                lax.fori_loop(0, HEADS, head, 0)
                return 0
            lax.fori_loop(0, nch, chunk, 0)

            def norm(h, _):
                dsum = jnp.sum(den_v[h, :]) + 1e-16
                rinv = 1.0 / jnp.full((16,), dsum, jnp.float32)

                def nf(c, _):
                    sl = pl.ds(h * HID + c * 16, 16)
                    acc_v[sl] = acc_v[sl] * rinv
                    return 0
                lax.fori_loop(0, HID // 16, nf, 0)
                return 0
            lax.fori_loop(0, HEADS, norm, 0)

            pltpu.sync_copy(acc_v, gat_hbm.at[base + rr])
            return 0
        lax.fori_loop(0, 16, node, 0)
        return 0
    lax.fori_loop(0, nn // 16, group, 0)


def _bn(x, g, b):
    mu = jnp.mean(x, axis=0)
    var = jnp.var(x, axis=0)
    return (x - mu) * jax.lax.rsqrt(var + 1e-5) * g + b


def _post_kernel(gat_ref, gat_b_ref, g_bn_ref, be_bn_ref, W_p1_ref, b_p1_ref,
                 g_p_ref, be_p_ref, W_p2_ref, b_p2_ref, out_ref):
    gat = gat_ref[...] + gat_b_ref[...]
    h = jax.nn.relu(_bn(gat, g_bn_ref[...], be_bn_ref[...]))
    h2 = jax.nn.relu(_bn(h @ W_p1_ref[...] + b_p1_ref[...], g_p_ref[...], be_p_ref[...]))
    out_ref[...] = (h2 @ W_p2_ref[...] + b_p2_ref[...])


def kernel(x, edge_index, edge_attr, W_atom, b_atom, W_edge, b_edge, W_msg, b_msg, g_msg, be_msg, W_l, b_l, W_r, b_r, att, gat_b, g_bn, be_bn, W_p1, b_p1, g_p, be_p, W_p2, b_p2):
    src = edge_index[0]
    dst = edge_index[1]

    hists, lrank, sege2 = _a1(dst, edge_attr)
    bases, segs = _a2(hists)
    ss2 = _a3(dst, src, lrank, bases, segs)

    srcs = ss2[0] + ss2[1]
    sege = (sege2[0] + sege2[1]).reshape(N, D_EDGE)
    cnt = hists[:, :N].sum(0).astype(jnp.float32)

    atom = x @ W_atom + b_atom
    agg = (sege @ W_edge + cnt[:, None] * b_edge) / jnp.maximum(cnt, 1.0)[:, None]
    msg = jax.nn.relu(_bn((atom + agg) @ W_msg + b_msg, g_msg, be_msg))
    comb = jnp.concatenate([msg, agg], axis=1)
    xl = comb @ W_l + b_l
    xr = comb @ W_r + b_r

    gat = _a4(xl, xr, srcs, segs, att.reshape(-1))

    out2 = pl.pallas_call(
        _post_kernel,
        out_shape=jax.ShapeDtypeStruct((N, 1), jnp.float32),
    )(gat, gat_b, g_bn, be_bn, W_p1, b_p1, g_p, be_p, W_p2, b_p2)
    return out2[:, 0]


# speedup vs baseline: 24.0692x; 1.1656x over previous
import functools

import jax
import jax.numpy as jnp
from jax import lax
from jax.experimental import pallas as pl
from jax.experimental.pallas import tpu as pltpu
from jax.experimental.pallas import tpu_sc as plsc

N = 10000
E = 320000
D_ATOM = 128
D_EDGE = 16
HID = 64
HEADS = 8

NC = 2
NS = 16
TILES = NC * NS
EPT = E // TILES
EN = E + N
ES_PAD = 333056
NP_PAD = 10240
SEGS_PAD = NP_PAD + 16 * 9
NPT = 312
SLP = 336
EA_BLK = 2000
DL = HEADS * HID
CAP = 12288
NTA = 320
NTB = 304

_mesh = plsc.VectorSubcoreMesh(core_axis_name="c", subcore_axis_name="s")
_sc_params = pltpu.CompilerParams(needs_layout_passes=False,
                                  use_tc_tiling_on_sc=False)

_DNUMS = lax.GatherDimensionNumbers(
    offset_dims=(), collapsed_slice_dims=(0,), start_index_map=(0,))


def _permute(x, idx):
    return lax.gather(x, idx[:, None], dimension_numbers=_DNUMS,
                      slice_sizes=(1,),
                      mode=lax.GatherScatterMode.PROMISE_IN_BOUNDS)


def _occ16(d16):
    lanes = lax.iota(jnp.int32, 16)
    occ = jnp.zeros((16,), jnp.int32)
    sh = d16
    for s in range(1, 16):
        sh = _permute(sh, jnp.maximum(lanes - 1, 0))
        occ = occ + jnp.where((sh == d16) & (lanes >= s), 1, 0)
    return occ


def _wid():
    return lax.axis_index("s") * NC + lax.axis_index("c")


@functools.partial(
    pl.kernel,
    out_type=(
        jax.ShapeDtypeStruct((TILES, NP_PAD), jnp.int32),
        jax.ShapeDtypeStruct((E,), jnp.int32),
        jax.ShapeDtypeStruct((NC, NS, N // NS, D_EDGE), jnp.float32),
    ),
    mesh=_mesh,
    scratch_types=(
        pltpu.VMEM((EPT,), jnp.int32),
        pltpu.VMEM((NP_PAD,), jnp.int32),
        pltpu.VMEM((EPT,), jnp.int32),
        pltpu.VMEM((EA_BLK, D_EDGE), jnp.float32),
        pltpu.VMEM((EA_BLK,), jnp.int32),
        pltpu.VMEM((N // NS, D_EDGE), jnp.float32),
        pltpu.VMEM_SHARED((N, D_EDGE), jnp.float32),
    ),
    compiler_params=_sc_params,
)
def _a1(dst_hbm, ea_hbm, hists_hbm, lrank_hbm, sege_hbm,
        dst_v, hist_v, lrank_v, ea_v, dstb_v, zb_v, sege_sh):
    wid = _wid()
    cid = lax.axis_index("c")
    sid = lax.axis_index("s")
    base = wid * EPT
    rows = N // NS

    def zrow(i, _):
        zb_v[i] = jnp.zeros((D_EDGE,), jnp.float32)
        return 0
    lax.fori_loop(0, rows, zrow, 0)
    pltpu.sync_copy(zb_v, sege_sh.at[pl.ds(sid * rows, rows)])

    def zhist(i, _):
        hist_v[pl.ds(i * 16, 16)] = jnp.zeros((16,), jnp.int32)
        return 0
    lax.fori_loop(0, NP_PAD // 16, zhist, 0)

    pltpu.sync_copy(dst_hbm.at[pl.ds(base, EPT)], dst_v)

    def body(i, _):
        sl = pl.ds(i * 16, 16)
        d16 = dst_v[sl]
        occ = _occ16(d16)
        c16 = plsc.load_gather(hist_v, [d16])
        lrank_v[sl] = c16 + occ
        plsc.addupdate_scatter(hist_v, [d16], jnp.ones((16,), jnp.int32))
        return 0
    lax.fori_loop(0, EPT // 16, body, 0)

    pltpu.sync_copy(hist_v, hists_hbm.at[wid])
    pltpu.sync_copy(lrank_v, lrank_hbm.at[pl.ds(base, EPT)])

    plsc.subcore_barrier()
    for b in range(EPT // EA_BLK):
        off = base + b * EA_BLK
        pltpu.sync_copy(ea_hbm.at[pl.ds(off, EA_BLK)], ea_v)
        pltpu.sync_copy(dst_hbm.at[pl.ds(off, EA_BLK)], dstb_v)
        pltpu.sync_copy(ea_v, sege_sh.at[dstb_v], add=True)
    plsc.subcore_barrier()

    pltpu.sync_copy(sege_sh.at[pl.ds(sid * rows, rows)], zb_v)
    pltpu.sync_copy(zb_v, sege_hbm.at[cid, sid])


_CH = 1024


@functools.partial(
    pl.kernel,
    out_type=(
        jax.ShapeDtypeStruct((TILES, NP_PAD), jnp.int32),
        jax.ShapeDtypeStruct((SEGS_PAD,), jnp.int32),
    ),
    mesh=_mesh,
    scratch_types=(
        pltpu.VMEM((TILES, _CH), jnp.int32),
        pltpu.VMEM((TILES, _CH), jnp.int32),
        pltpu.VMEM((_CH,), jnp.int32),
        pltpu.VMEM((16,), jnp.int32),
    ),
    compiler_params=_sc_params,
)
def _a2(hists_hbm, bases_hbm, segs_hbm, hcol_v, bcol_v, seg_v, pad_v):
    wid = _wid()

    @pl.when(wid == 0)
    def _():
        def chunk(ci, carry0):
            c0 = ci * _CH
            pltpu.sync_copy(hists_hbm.at[:, pl.ds(c0, _CH)], hcol_v)

            def step(j, carry_in):
                sl = pl.ds(j * 16, 16)
                tot = jnp.ones((16,), jnp.int32)
                for t in range(TILES):
                    tot = tot + hcol_v[t, sl]
                incl = plsc.cumsum(tot)
                seg = incl - tot + carry_in
                seg_v[sl] = seg
                b = seg + 1
                for t in range(TILES):
                    bcol_v[t, sl] = b
                    b = b + hcol_v[t, sl]
                return carry_in + jnp.sum(tot)

            carry1 = lax.fori_loop(0, _CH // 16, step, carry0)
            pltpu.sync_copy(bcol_v, bases_hbm.at[:, pl.ds(c0, _CH)])
            pltpu.sync_copy(seg_v, segs_hbm.at[pl.ds(c0, _CH)])
            return carry1

        lax.fori_loop(0, NP_PAD // _CH, chunk, jnp.int32(0))

        def pad(i, _):
            pad_v[...] = jnp.full((16,), EN, jnp.int32)
            pltpu.sync_copy(pad_v, segs_hbm.at[pl.ds(N + i * 16, 16)])
            return 0
        lax.fori_loop(0, (SEGS_PAD - N) // 16, pad, 0)


@functools.partial(
    pl.kernel,
    out_type=jax.ShapeDtypeStruct((NC, ES_PAD), jnp.int32),
    mesh=_mesh,
    scratch_types=(
        pltpu.VMEM((EPT,), jnp.int32),
        pltpu.VMEM((EPT,), jnp.int32),
        pltpu.VMEM((NP_PAD,), jnp.int32),
        pltpu.VMEM((EPT,), jnp.int32),
        pltpu.VMEM((EPT,), jnp.int32),
        pltpu.VMEM((SLP,), jnp.int32),
        pltpu.VMEM((SLP,), jnp.int32),
        pltpu.VMEM((ES_PAD // NS,), jnp.int32),
        pltpu.VMEM_SHARED((ES_PAD,), jnp.int32),
    ),
    compiler_params=_sc_params,
)
def _a3(dst_hbm, src_hbm, lrank_hbm, bases_hbm, segs_hbm, out_hbm,
        dst_v, lrank_v, bases_v, src_v, pos_v, spos_v, sval_v, zb_v,
        sorted_sh):
    wid = _wid()
    cid = lax.axis_index("c")
    sid = lax.axis_index("s")
    base = wid * EPT
    zwords = ES_PAD // NS

    def zrow(i, _):
        zb_v[pl.ds(i * 16, 16)] = jnp.zeros((16,), jnp.int32)
        return 0
    lax.fori_loop(0, zwords // 16, zrow, 0)
    pltpu.sync_copy(zb_v, sorted_sh.at[pl.ds(sid * zwords, zwords)])

    pltpu.sync_copy(dst_hbm.at[pl.ds(base, EPT)], dst_v)
    pltpu.sync_copy(lrank_hbm.at[pl.ds(base, EPT)], lrank_v)
    pltpu.sync_copy(bases_hbm.at[wid], bases_v)
    pltpu.sync_copy(src_hbm.at[pl.ds(base, EPT)], src_v)

    def mkpos(i, _):
        sl = pl.ds(i * 16, 16)
        d16 = dst_v[sl]
        b16 = plsc.load_gather(bases_v, [d16])
        pos_v[sl] = b16 + lrank_v[sl]
        return 0
    lax.fori_loop(0, EPT // 16, mkpos, 0)

    n0 = wid * NPT
    hi = jnp.where(wid == TILES - 1, N, n0 + NPT)
    pltpu.sync_copy(segs_hbm.at[pl.ds(n0, SLP)], spos_v)

    def mkself(i, _):
        sl = pl.ds(i * 16, 16)
        node = lax.iota(jnp.int32, 16) + (n0 + i * 16)
        ok = node < hi
        sval_v[sl] = jnp.where(ok, node, 0)
        spos_v[sl] = jnp.where(ok, spos_v[sl], EN)
        return 0
    lax.fori_loop(0, SLP // 16, mkself, 0)

    plsc.subcore_barrier()
    pltpu.sync_copy(src_v, sorted_sh.at[pos_v], add=True)
    pltpu.sync_copy(sval_v, sorted_sh.at[spos_v], add=True)
    plsc.subcore_barrier()

    pltpu.sync_copy(sorted_sh.at[pl.ds(sid * zwords, zwords)], zb_v)
    pltpu.sync_copy(zb_v, out_hbm.at[cid, pl.ds(sid * zwords, zwords)])


@functools.partial(
    pl.kernel,
    out_type=jax.ShapeDtypeStruct((N, DL), jnp.float32),
    mesh=_mesh,
    scratch_types=(
        pltpu.VMEM((SEGS_PAD,), jnp.int32),
        pltpu.VMEM((CAP,), jnp.int32),
        pltpu.VMEM((DL,), jnp.float32),
        pltpu.VMEM((16, DL), jnp.float32),
        pltpu.VMEM((32, DL), jnp.float32),
        pltpu.VMEM((DL,), jnp.float32),
        pltpu.VMEM((HEADS, 16), jnp.float32),
        pltpu.SemaphoreType.DMA,
    ),
    compiler_params=_sc_params,
)
def _a4(xl_hbm, xr_hbm, srcs_hbm, segs_hbm, att_hbm, gat_hbm,
        seg_v, span_v, att_v, xrg_v, rows_v, acc_v, den_v, sem):
    wid = _wid()
    n0 = jnp.where(wid < 17, wid * NTA, 17 * NTA + (wid - 17) * NTB)
    nn = jnp.where(wid < 17, NTA, NTB)
    lanes = lax.iota(jnp.int32, 16)

    pltpu.sync_copy(segs_hbm, seg_v)
    pltpu.sync_copy(att_hbm, att_v)
    e0 = seg_v[pl.ds(n0, 16)][0]
    ebase = jnp.minimum((e0 // 8) * 8, ES_PAD - CAP)

    def cpspan(k, _):
        pltpu.sync_copy(srcs_hbm.at[pl.ds(ebase + k * 2048, 2048)],
                        span_v.at[pl.ds(k * 2048, 2048)])
        return 0
    lax.fori_loop(0, CAP // 2048, cpspan, 0)

    def group(gi, _):
        base = n0 + gi * 16
        sva = seg_v[pl.ds(base, 16)]
        send = seg_v[pl.ds(base + 8, 16)][8]
        pltpu.sync_copy(xr_hbm.at[pl.ds(base, 16)], xrg_v)

        def node(rr, _):
            cur = jnp.sum(jnp.where(lanes == rr, sva, 0))
            nxt = jnp.where(rr == 15, send,
                            jnp.sum(jnp.where(lanes == rr + 1, sva, 0)))

            def zc(c, _):
                acc_v[pl.ds(c * 16, 16)] = jnp.zeros((16,), jnp.float32)
                return 0
            lax.fori_loop(0, DL // 16, zc, 0)

            def zd(h, _):
                den_v[h, :] = jnp.zeros((16,), jnp.float32)
                return 0
            lax.fori_loop(0, HEADS, zd, 0)

            nch = (jnp.minimum(nxt - cur, CAP) + 15) // 16

            def _eidx(ch):
                pos = cur + ch * 16 + lanes
                valid = pos < nxt
                rel = jnp.clip(jnp.where(valid, pos, nxt - 1) - ebase,
                               0, CAP - 1)
                return plsc.load_gather(span_v, [rel]), valid

            idx0, _ = _eidx(0)
            pltpu.async_copy(xl_hbm.at[idx0], rows_v.at[pl.ds(0, 16)], sem)

            def chunk(ch, _):
                rb = (ch % 2) * 16
                idx16, valid = _eidx(ch)
                pltpu.make_async_copy(xl_hbm.at[idx16],
                                      rows_v.at[pl.ds(rb, 16)], sem).wait()

                @pl.when(ch + 1 < nch)
                def _():
                    idxn, _v = _eidx(ch + 1)
                    pltpu.async_copy(xl_hbm.at[idxn],
                                     rows_v.at[pl.ds((rb + 16) % 32, 16)],
                                     sem)

                def head(h, _):
                    sls = [pl.ds(h * HID + c * 16, 16) for c in range(HID // 16)]
                    xrh = [xrg_v[rr, sl] for sl in sls]
                    a6 = [att_v[sl] * 0.6 for sl in sls]
                    a4 = [att_v[sl] * 0.4 for sl in sls]
                    logit = jnp.zeros((16,), jnp.float32)
                    for r in range(16):
                        p = jnp.zeros((16,), jnp.float32)
                        for c, sl in enumerate(sls):
                            t = rows_v[rb + r, sl] + xrh[c]
                            p = p + t * a6[c] + jnp.abs(t) * a4[c]
                        logit = logit + jnp.where(lanes == r, jnp.sum(p), 0.0)
                    w = jnp.where(valid, jnp.exp(logit), 0.0)
                    den_v[h, :] = den_v[h, :] + w
                    for c, sl in enumerate(sls):
                        a = acc_v[sl]
                        for r in range(16):
                            a = a + w[r] * rows_v[rb + r, sl]
                        acc_v[sl] = a
                    return 0
                lax.fori_loop(0, HEADS, head, 0)
                return 0
            lax.fori_loop(0, nch, chunk, 0)

            def norm(h, _):
                dsum = jnp.sum(den_v[h, :]) + 1e-16
                rinv = 1.0 / jnp.full((16,), dsum, jnp.float32)

                def nf(c, _):
                    sl = pl.ds(h * HID + c * 16, 16)
                    acc_v[sl] = acc_v[sl] * rinv
                    return 0
                lax.fori_loop(0, HID // 16, nf, 0)
                return 0
            lax.fori_loop(0, HEADS, norm, 0)

            pltpu.sync_copy(acc_v, gat_hbm.at[base + rr])
            return 0
        lax.fori_loop(0, 16, node, 0)
        return 0
    lax.fori_loop(0, nn // 16, group, 0)


def _bn(x, g, b):
    mu = jnp.mean(x, axis=0)
    var = jnp.var(x, axis=0)
    return (x - mu) * jax.lax.rsqrt(var + 1e-5) * g + b


def _post_kernel(gat_ref, gat_b_ref, g_bn_ref, be_bn_ref, W_p1_ref, b_p1_ref,
                 g_p_ref, be_p_ref, W_p2_ref, b_p2_ref, out_ref):
    gat = gat_ref[...] + gat_b_ref[...]
    h = jax.nn.relu(_bn(gat, g_bn_ref[...], be_bn_ref[...]))
    h2 = jax.nn.relu(_bn(h @ W_p1_ref[...] + b_p1_ref[...], g_p_ref[...], be_p_ref[...]))
    out_ref[...] = (h2 @ W_p2_ref[...] + b_p2_ref[...])


def kernel(x, edge_index, edge_attr, W_atom, b_atom, W_edge, b_edge, W_msg, b_msg, g_msg, be_msg, W_l, b_l, W_r, b_r, att, gat_b, g_bn, be_bn, W_p1, b_p1, g_p, be_p, W_p2, b_p2):
    src = edge_index[0]
    dst = edge_index[1]

    hists, lrank, sege2 = _a1(dst, edge_attr)
    bases, segs = _a2(hists)
    ss2 = _a3(dst, src, lrank, bases, segs)

    srcs = ss2[0] + ss2[1]
    sege = (sege2[0] + sege2[1]).reshape(N, D_EDGE)
    cnt = hists[:, :N].sum(0).astype(jnp.float32)

    atom = x @ W_atom + b_atom
    agg = (sege @ W_edge + cnt[:, None] * b_edge) / jnp.maximum(cnt, 1.0)[:, None]
    msg = jax.nn.relu(_bn((atom + agg) @ W_msg + b_msg, g_msg, be_msg))
    comb = jnp.concatenate([msg, agg], axis=1)
    xl = comb @ W_l + b_l
    xr = comb @ W_r + b_r

    gat = _a4(xl, xr, srcs, segs, att.reshape(-1))

    out2 = pl.pallas_call(
        _post_kernel,
        out_shape=jax.ShapeDtypeStruct((N, 1), jnp.float32),
    )(gat, gat_b, g_bn, be_bn, W_p1, b_p1, g_p, be_p, W_p2, b_p2)
    return out2[:, 0]


# speedup vs baseline: 24.2930x; 1.0093x over previous
import functools

import jax
import jax.numpy as jnp
from jax import lax
from jax.experimental import pallas as pl
from jax.experimental.pallas import tpu as pltpu
from jax.experimental.pallas import tpu_sc as plsc

N = 10000
E = 320000
D_ATOM = 128
D_EDGE = 16
HID = 64
HEADS = 8

NC = 2
NS = 16
TILES = NC * NS
EPT = E // TILES
EN = E + N
ES_PAD = 333056
NP_PAD = 10240
SEGS_PAD = NP_PAD + 16 * 9
NPT = 312
SLP = 336
EA_BLK = 2000
DL = HEADS * HID
CAP = 12288
NTA = 320
NTB = 304

_mesh = plsc.VectorSubcoreMesh(core_axis_name="c", subcore_axis_name="s")
_sc_params = pltpu.CompilerParams(needs_layout_passes=False,
                                  use_tc_tiling_on_sc=False)

_DNUMS = lax.GatherDimensionNumbers(
    offset_dims=(), collapsed_slice_dims=(0,), start_index_map=(0,))


def _permute(x, idx):
    return lax.gather(x, idx[:, None], dimension_numbers=_DNUMS,
                      slice_sizes=(1,),
                      mode=lax.GatherScatterMode.PROMISE_IN_BOUNDS)


def _occ16(d16):
    lanes = lax.iota(jnp.int32, 16)
    occ = jnp.zeros((16,), jnp.int32)
    sh = d16
    for s in range(1, 16):
        sh = _permute(sh, jnp.maximum(lanes - 1, 0))
        occ = occ + jnp.where((sh == d16) & (lanes >= s), 1, 0)
    return occ


def _wid():
    return lax.axis_index("s") * NC + lax.axis_index("c")


@functools.partial(
    pl.kernel,
    out_type=(
        jax.ShapeDtypeStruct((TILES, NP_PAD), jnp.int32),
        jax.ShapeDtypeStruct((E,), jnp.int32),
        jax.ShapeDtypeStruct((NC, NS, N // NS, D_EDGE), jnp.float32),
    ),
    mesh=_mesh,
    scratch_types=(
        pltpu.VMEM((EPT,), jnp.int32),
        pltpu.VMEM((NP_PAD,), jnp.int32),
        pltpu.VMEM((EPT,), jnp.int32),
        pltpu.VMEM((EA_BLK, D_EDGE), jnp.float32),
        pltpu.VMEM((EA_BLK,), jnp.int32),
        pltpu.VMEM((N // NS, D_EDGE), jnp.float32),
        pltpu.VMEM_SHARED((N, D_EDGE), jnp.float32),
    ),
    compiler_params=_sc_params,
)
def _a1(dst_hbm, ea_hbm, hists_hbm, lrank_hbm, sege_hbm,
        dst_v, hist_v, lrank_v, ea_v, dstb_v, zb_v, sege_sh):
    wid = _wid()
    cid = lax.axis_index("c")
    sid = lax.axis_index("s")
    base = wid * EPT
    rows = N // NS

    def zrow(i, _):
        zb_v[i] = jnp.zeros((D_EDGE,), jnp.float32)
        return 0
    lax.fori_loop(0, rows, zrow, 0)
    pltpu.sync_copy(zb_v, sege_sh.at[pl.ds(sid * rows, rows)])

    def zhist(i, _):
        hist_v[pl.ds(i * 16, 16)] = jnp.zeros((16,), jnp.int32)
        return 0
    lax.fori_loop(0, NP_PAD // 16, zhist, 0)

    pltpu.sync_copy(dst_hbm.at[pl.ds(base, EPT)], dst_v)

    def body(i, _):
        sl = pl.ds(i * 16, 16)
        d16 = dst_v[sl]
        occ = _occ16(d16)
        c16 = plsc.load_gather(hist_v, [d16])
        lrank_v[sl] = c16 + occ
        plsc.addupdate_scatter(hist_v, [d16], jnp.ones((16,), jnp.int32))
        return 0
    lax.fori_loop(0, EPT // 16, body, 0)

    pltpu.sync_copy(hist_v, hists_hbm.at[wid])
    pltpu.sync_copy(lrank_v, lrank_hbm.at[pl.ds(base, EPT)])

    plsc.subcore_barrier()
    for b in range(EPT // EA_BLK):
        off = base + b * EA_BLK
        pltpu.sync_copy(ea_hbm.at[pl.ds(off, EA_BLK)], ea_v)
        pltpu.sync_copy(dst_hbm.at[pl.ds(off, EA_BLK)], dstb_v)
        pltpu.sync_copy(ea_v, sege_sh.at[dstb_v], add=True)
    plsc.subcore_barrier()

    pltpu.sync_copy(sege_sh.at[pl.ds(sid * rows, rows)], zb_v)
    pltpu.sync_copy(zb_v, sege_hbm.at[cid, sid])


_CH = 1024


@functools.partial(
    pl.kernel,
    out_type=(
        jax.ShapeDtypeStruct((TILES, NP_PAD), jnp.int32),
        jax.ShapeDtypeStruct((SEGS_PAD,), jnp.int32),
    ),
    mesh=_mesh,
    scratch_types=(
        pltpu.VMEM((TILES, _CH), jnp.int32),
        pltpu.VMEM((TILES, _CH), jnp.int32),
        pltpu.VMEM((_CH,), jnp.int32),
        pltpu.VMEM((16,), jnp.int32),
    ),
    compiler_params=_sc_params,
)
def _a2(hists_hbm, bases_hbm, segs_hbm, hcol_v, bcol_v, seg_v, pad_v):
    wid = _wid()

    @pl.when(wid == 0)
    def _():
        def chunk(ci, carry0):
            c0 = ci * _CH
            pltpu.sync_copy(hists_hbm.at[:, pl.ds(c0, _CH)], hcol_v)

            def step(j, carry_in):
                sl = pl.ds(j * 16, 16)
                tot = jnp.ones((16,), jnp.int32)
                for t in range(TILES):
                    tot = tot + hcol_v[t, sl]
                incl = plsc.cumsum(tot)
                seg = incl - tot + carry_in
                seg_v[sl] = seg
                b = seg + 1
                for t in range(TILES):
                    bcol_v[t, sl] = b
                    b = b + hcol_v[t, sl]
                return carry_in + jnp.sum(tot)

            carry1 = lax.fori_loop(0, _CH // 16, step, carry0)
            pltpu.sync_copy(bcol_v, bases_hbm.at[:, pl.ds(c0, _CH)])
            pltpu.sync_copy(seg_v, segs_hbm.at[pl.ds(c0, _CH)])
            return carry1

        lax.fori_loop(0, NP_PAD // _CH, chunk, jnp.int32(0))

        def pad(i, _):
            pad_v[...] = jnp.full((16,), EN, jnp.int32)
            pltpu.sync_copy(pad_v, segs_hbm.at[pl.ds(N + i * 16, 16)])
            return 0
        lax.fori_loop(0, (SEGS_PAD - N) // 16, pad, 0)


@functools.partial(
    pl.kernel,
    out_type=jax.ShapeDtypeStruct((NC, ES_PAD), jnp.int32),
    mesh=_mesh,
    scratch_types=(
        pltpu.VMEM((EPT,), jnp.int32),
        pltpu.VMEM((EPT,), jnp.int32),
        pltpu.VMEM((NP_PAD,), jnp.int32),
        pltpu.VMEM((EPT,), jnp.int32),
        pltpu.VMEM((EPT,), jnp.int32),
        pltpu.VMEM((SLP,), jnp.int32),
        pltpu.VMEM((SLP,), jnp.int32),
        pltpu.VMEM((ES_PAD // NS,), jnp.int32),
        pltpu.VMEM_SHARED((ES_PAD,), jnp.int32),
    ),
    compiler_params=_sc_params,
)
def _a3(dst_hbm, src_hbm, lrank_hbm, bases_hbm, segs_hbm, out_hbm,
        dst_v, lrank_v, bases_v, src_v, pos_v, spos_v, sval_v, zb_v,
        sorted_sh):
    wid = _wid()
    cid = lax.axis_index("c")
    sid = lax.axis_index("s")
    base = wid * EPT
    zwords = ES_PAD // NS

    def zrow(i, _):
        zb_v[pl.ds(i * 16, 16)] = jnp.zeros((16,), jnp.int32)
        return 0
    lax.fori_loop(0, zwords // 16, zrow, 0)
    pltpu.sync_copy(zb_v, sorted_sh.at[pl.ds(sid * zwords, zwords)])

    pltpu.sync_copy(dst_hbm.at[pl.ds(base, EPT)], dst_v)
    pltpu.sync_copy(lrank_hbm.at[pl.ds(base, EPT)], lrank_v)
    pltpu.sync_copy(bases_hbm.at[wid], bases_v)
    pltpu.sync_copy(src_hbm.at[pl.ds(base, EPT)], src_v)

    def mkpos(i, _):
        sl = pl.ds(i * 16, 16)
        d16 = dst_v[sl]
        b16 = plsc.load_gather(bases_v, [d16])
        pos_v[sl] = b16 + lrank_v[sl]
        return 0
    lax.fori_loop(0, EPT // 16, mkpos, 0)

    n0 = wid * NPT
    hi = jnp.where(wid == TILES - 1, N, n0 + NPT)
    pltpu.sync_copy(segs_hbm.at[pl.ds(n0, SLP)], spos_v)

    def mkself(i, _):
        sl = pl.ds(i * 16, 16)
        node = lax.iota(jnp.int32, 16) + (n0 + i * 16)
        ok = node < hi
        sval_v[sl] = jnp.where(ok, node, 0)
        spos_v[sl] = jnp.where(ok, spos_v[sl], EN)
        return 0
    lax.fori_loop(0, SLP // 16, mkself, 0)

    plsc.subcore_barrier()
    pltpu.sync_copy(src_v, sorted_sh.at[pos_v], add=True)
    pltpu.sync_copy(sval_v, sorted_sh.at[spos_v], add=True)
    plsc.subcore_barrier()

    pltpu.sync_copy(sorted_sh.at[pl.ds(sid * zwords, zwords)], zb_v)
    pltpu.sync_copy(zb_v, out_hbm.at[cid, pl.ds(sid * zwords, zwords)])


@functools.partial(
    pl.kernel,
    out_type=jax.ShapeDtypeStruct((N, DL), jnp.float32),
    mesh=_mesh,
    scratch_types=(
        pltpu.VMEM((SEGS_PAD,), jnp.int32),
        pltpu.VMEM((CAP,), jnp.int32),
        pltpu.VMEM((DL,), jnp.float32),
        pltpu.VMEM((16, DL), jnp.float32),
        pltpu.VMEM((32, DL), jnp.float32),
        pltpu.VMEM((DL,), jnp.float32),
        pltpu.VMEM((HEADS, 16), jnp.float32),
        pltpu.SemaphoreType.DMA,
    ),
    compiler_params=_sc_params,
)
def _a4(xl_hbm, xr_hbm, srcs_hbm, segs_hbm, att_hbm, gat_hbm,
        seg_v, span_v, att_v, xrg_v, rows_v, acc_v, den_v, sem):
    wid = _wid()
    n0 = jnp.where(wid < 17, wid * NTA, 17 * NTA + (wid - 17) * NTB)
    nn = jnp.where(wid < 17, NTA, NTB)
    lanes = lax.iota(jnp.int32, 16)

    pltpu.sync_copy(segs_hbm, seg_v)
    pltpu.sync_copy(att_hbm, att_v)
    e0 = seg_v[pl.ds(n0, 16)][0]
    ebase = jnp.minimum((e0 // 8) * 8, ES_PAD - CAP)

    def cpspan(k, _):
        pltpu.sync_copy(srcs_hbm.at[pl.ds(ebase + k * 2048, 2048)],
                        span_v.at[pl.ds(k * 2048, 2048)])
        return 0
    lax.fori_loop(0, CAP // 2048, cpspan, 0)

    def group(gi, _):
        base = n0 + gi * 16
        sva = seg_v[pl.ds(base, 16)]
        send = seg_v[pl.ds(base + 8, 16)][8]
        pltpu.sync_copy(xr_hbm.at[pl.ds(base, 16)], xrg_v)

        def node(rr, _):
            cur = jnp.sum(jnp.where(lanes == rr, sva, 0))
            nxt = jnp.where(rr == 15, send,
                            jnp.sum(jnp.where(lanes == rr + 1, sva, 0)))

            def zc(c, _):
                acc_v[pl.ds(c * 16, 16)] = jnp.zeros((16,), jnp.float32)
                return 0
            lax.fori_loop(0, DL // 16, zc, 0)

            def zd(h, _):
                den_v[h, :] = jnp.zeros((16,), jnp.float32)
                return 0
            lax.fori_loop(0, HEADS, zd, 0)

            nch = (jnp.minimum(nxt - cur, CAP) + 15) // 16

            def _eidx(ch):
                pos = cur + ch * 16 + lanes
                valid = pos < nxt
                rel = jnp.clip(jnp.where(valid, pos, nxt - 1) - ebase,
                               0, CAP - 1)
                return plsc.load_gather(span_v, [rel]), valid

            idx0, _ = _eidx(0)
            pltpu.async_copy(xl_hbm.at[idx0], rows_v.at[pl.ds(0, 16)], sem)

            def chunk(ch, _):
                rb = (ch % 2) * 16
                idx16, valid = _eidx(ch)
                pltpu.make_async_copy(xl_hbm.at[idx16],
                                      rows_v.at[pl.ds(rb, 16)], sem).wait()

                @pl.when(ch + 1 < nch)
                def _():
                    idxn, _v = _eidx(ch + 1)
                    pltpu.async_copy(xl_hbm.at[idxn],
                                     rows_v.at[pl.ds((rb + 16) % 32, 16)],
                                     sem)

                def head(h, _):
                    sls = [pl.ds(h * HID + c * 16, 16) for c in range(HID // 16)]
                    xrh = [xrg_v[rr, sl] for sl in sls]
                    a6 = [att_v[sl] * 0.6 for sl in sls]
                    a4 = [att_v[sl] * 0.4 for sl in sls]
                    logit = jnp.zeros((16,), jnp.float32)
                    for r in range(16):
                        p = jnp.zeros((16,), jnp.float32)
                        for c, sl in enumerate(sls):
                            t = rows_v[rb + r, sl] + xrh[c]
                            p = p + t * a6[c] + jnp.abs(t) * a4[c]
                        logit = logit + jnp.where(lanes == r, jnp.sum(p), 0.0)
                    w = jnp.where(valid, jnp.exp(logit), 0.0)
                    den_v[h, :] = den_v[h, :] + w
                    for c, sl in enumerate(sls):
                        a = acc_v[sl]
                        for r in range(16):
                            a = a + w[r] * rows_v[rb + r, sl]
                        acc_v[sl] = a
                    return 0
                lax.fori_loop(0, HEADS, head, 0)
                return 0
            lax.fori_loop(0, nch, chunk, 0)

            def norm(h, _):
                dsum = jnp.sum(den_v[h, :]) + 1e-16
                rinv = 1.0 / jnp.full((16,), dsum, jnp.float32)

                def nf(c, _):
                    sl = pl.ds(h * HID + c * 16, 16)
                    acc_v[sl] = acc_v[sl] * rinv
                    return 0
                lax.fori_loop(0, HID // 16, nf, 0)
                return 0
            lax.fori_loop(0, HEADS, norm, 0)

            pltpu.sync_copy(acc_v, gat_hbm.at[base + rr])
            return 0
        lax.fori_loop(0, 16, node, 0)
        return 0
    lax.fori_loop(0, nn // 16, group, 0)


def _bn(x, g, b):
    mu = jnp.mean(x, axis=0)
    var = jnp.var(x, axis=0)
    return (x - mu) * jax.lax.rsqrt(var + 1e-5) * g + b


def _pre1_kernel(x_ref, W_atom_ref, b_atom_ref, sege2_ref, W_edge_ref,
                 b_edge_ref, hists_ref, W_msg_ref, b_msg_ref, g_msg_ref,
                 be_msg_ref, comb_ref):
    atom = x_ref[...] @ W_atom_ref[...] + b_atom_ref[...]
    cnt = jnp.sum(hists_ref[...][:, :N], axis=0).astype(jnp.float32)
    sege = sege2_ref[...][0] + sege2_ref[...][1]
    agg = ((sege @ W_edge_ref[...] + cnt[:, None] * b_edge_ref[...])
           / jnp.maximum(cnt, 1.0)[:, None])
    msg = jax.nn.relu(_bn((atom + agg) @ W_msg_ref[...] + b_msg_ref[...],
                          g_msg_ref[...], be_msg_ref[...]))
    comb_ref[...] = jnp.concatenate([msg, agg], axis=1)


def _mm_kernel(a_ref, w_ref, b_ref, o_ref):
    o_ref[...] = a_ref[...] @ w_ref[...] + b_ref[...]


def _post_kernel(gat_ref, gat_b_ref, g_bn_ref, be_bn_ref, W_p1_ref, b_p1_ref,
                 g_p_ref, be_p_ref, W_p2_ref, b_p2_ref, out_ref):
    gat = gat_ref[...] + gat_b_ref[...]
    h = jax.nn.relu(_bn(gat, g_bn_ref[...], be_bn_ref[...]))
    h2 = jax.nn.relu(_bn(h @ W_p1_ref[...] + b_p1_ref[...], g_p_ref[...], be_p_ref[...]))
    out_ref[...] = (h2 @ W_p2_ref[...] + b_p2_ref[...])


def kernel(x, edge_index, edge_attr, W_atom, b_atom, W_edge, b_edge, W_msg, b_msg, g_msg, be_msg, W_l, b_l, W_r, b_r, att, gat_b, g_bn, be_bn, W_p1, b_p1, g_p, be_p, W_p2, b_p2):
    src = edge_index[0]
    dst = edge_index[1]

    hists, lrank, sege2 = _a1(dst, edge_attr)
    bases, segs = _a2(hists)
    ss2 = _a3(dst, src, lrank, bases, segs)

    srcs = ss2[0] + ss2[1]
    sege2r = sege2.reshape(NC, N, D_EDGE)

    comb = pl.pallas_call(
        _pre1_kernel,
        out_shape=jax.ShapeDtypeStruct((N, 2 * HID), jnp.float32),
    )(x, W_atom, b_atom, sege2r, W_edge, b_edge, hists, W_msg, b_msg,
      g_msg, be_msg)
    mm = pl.pallas_call(
        _mm_kernel,
        out_shape=jax.ShapeDtypeStruct((N, HEADS * HID), jnp.float32),
    )
    xl = mm(comb, W_l, b_l)
    xr = mm(comb, W_r, b_r)

    gat = _a4(xl, xr, srcs, segs, att.reshape(-1))

    out2 = pl.pallas_call(
        _post_kernel,
        out_shape=jax.ShapeDtypeStruct((N, 1), jnp.float32),
    )(gat, gat_b, g_bn, be_bn, W_p1, b_p1, g_p, be_p, W_p2, b_p2)
    return out2[:, 0]
